# baseline (speedup 1.0000x reference)
import functools

import jax
import jax.numpy as jnp
from jax import lax
from jax.experimental import pallas as pl
from jax.experimental.pallas import tpu as pltpu
from jax.experimental.pallas import tpu_sc as plsc

_N = 10000
_E = 320000
_D = 128
_G = 64
_C = 16

_NC = 2
_NS = 16
_K = 128

_NPAD = 10112
_RPS = _NPAD // _NS

_EP = _E + _N
_NPH = 2
_NITER = 42
_EPAD = _NC * _NS * _NPH * _NITER * _K

_mesh = plsc.VectorSubcoreMesh(core_axis_name="c", subcore_axis_name="s")


@functools.partial(
    pl.kernel,
    out_type=jax.ShapeDtypeStruct((_NC, _NPAD, _D), jnp.float32),
    mesh=_mesh,
    scratch_types=[
        pltpu.VMEM((_NITER, _K), jnp.int32),
        pltpu.VMEM((_K, _D), jnp.float32),
        pltpu.VMEM_SHARED((_NPAD, _D), jnp.float32),
    ],
)
def _deg_kernel(dst_hbm, zeros_hbm, ones_hbm, out_hbm, dst_v, ones_v, acc_sh):
    c = lax.axis_index("c")
    s = lax.axis_index("s")
    rows = pl.ds(s * _RPS, _RPS)
    pltpu.sync_copy(zeros_hbm.at[rows], acc_sh.at[rows])
    pltpu.sync_copy(ones_hbm, ones_v)
    for ph in range(_NPH):
        pltpu.sync_copy(dst_hbm.at[c, s, ph], dst_v)
        if ph == 0:
            plsc.subcore_barrier()

        @pl.loop(0, _NITER)
        def _(j):
            pltpu.sync_copy(ones_v, acc_sh.at[dst_v.at[j]], add=True)

    plsc.subcore_barrier()
    pltpu.sync_copy(acc_sh.at[rows], out_hbm.at[c, rows])


@functools.partial(
    pl.kernel,
    out_type=jax.ShapeDtypeStruct((_NC, _NPAD, _D), jnp.float32),
    mesh=_mesh,
    scratch_types=[
        pltpu.VMEM((_NITER, _K), jnp.int32),
        pltpu.VMEM((_NITER, _K), jnp.int32),
        pltpu.VMEM((_K, _D), jnp.float32),
        pltpu.VMEM((_K, _D), jnp.float32),
        pltpu.VMEM_SHARED((_NPAD, _D), jnp.float32),
        pltpu.SemaphoreType.DMA,
        pltpu.SemaphoreType.DMA,
        pltpu.SemaphoreType.DMA,
        pltpu.SemaphoreType.DMA,
    ],
)
def _msg_kernel(hs_hbm, src_hbm, dst_hbm, zeros_hbm, out_hbm,
                src_v, dst_v, b0, b1, acc_sh, sg0, sg1, ss0, ss1):
    c = lax.axis_index("c")
    s = lax.axis_index("s")
    rows = pl.ds(s * _RPS, _RPS)
    pltpu.sync_copy(zeros_hbm.at[rows], acc_sh.at[rows])

    first = [True]

    def _phase(ph):
        pltpu.sync_copy(src_hbm.at[c, s, ph], src_v)
        pltpu.sync_copy(dst_hbm.at[c, s, ph], dst_v)
        if first[0]:
            plsc.subcore_barrier()
            first[0] = False
        @pl.loop(0, _NITER)
        def _(j):
            pltpu.sync_copy(hs_hbm.at[src_v.at[j]], b0)
            pltpu.sync_copy(b0, acc_sh.at[dst_v.at[j]], add=True)

    for ph in range(_NPH):
        _phase(ph)
    plsc.subcore_barrier()
    pltpu.sync_copy(acc_sh.at[rows], out_hbm.at[c, rows])


def _dinv_from_degp(degp):
    deg = degp[0, :, 0:1] + degp[1, :, 0:1]
    return lax.rsqrt(jnp.maximum(deg, 1.0))


def _tc_matmul_body(x_ref, w_ref, o_ref):
    o_ref[...] = jnp.dot(x_ref[...], w_ref[...],
                         preferred_element_type=jnp.float32)


_tc_matmul = pl.pallas_call(
    _tc_matmul_body,
    out_shape=jax.ShapeDtypeStruct((_NPAD, _D), jnp.float32),
)


def _tc_scale_body(hm_ref, degp_ref, o_ref):
    o_ref[...] = hm_ref[...] * _dinv_from_degp(degp_ref[...])


_tc_scale = pl.pallas_call(
    _tc_scale_body,
    out_shape=jax.ShapeDtypeStruct((_NPAD, _D), jnp.float32),
)


def _bn_relu(p_ref, degp_ref, b_ref, g_ref, be_ref):
    dinv = _dinv_from_degp(degp_ref[...])
    y = (p_ref[0] + p_ref[1]) * dinv + b_ref[...]
    mask = lax.broadcasted_iota(jnp.int32, (_NPAD, 1), 0) < _N
    ym = jnp.where(mask, y, 0.0)
    mu = jnp.sum(ym, axis=0, keepdims=True) * (1.0 / _N)
    d2 = jnp.where(mask, y - mu, 0.0)
    var = jnp.sum(d2 * d2, axis=0, keepdims=True) * (1.0 / _N)
    h = (y - mu) * lax.rsqrt(var + 1e-5) * g_ref[...] + be_ref[...]
    h = jnp.maximum(h, 0.0)
    return jnp.where(mask, h, 0.0), dinv


def _tc_layer_body(p_ref, degp_ref, b_ref, g_ref, be_ref, w_ref, o_ref):
    h, dinv = _bn_relu(p_ref, degp_ref, b_ref, g_ref, be_ref)
    o_ref[...] = jnp.dot(h * dinv, w_ref[...],
                         preferred_element_type=jnp.float32)


_tc_layer = pl.pallas_call(
    _tc_layer_body,
    out_shape=jax.ShapeDtypeStruct((_NPAD, _D), jnp.float32),
)


def _tc_head_body(p_ref, degp_ref, b_ref, g_ref, be_ref, batch_ref,
                  fw1_ref, fb1_ref, fw2_ref, fb2_ref, o_ref):
    h, _ = _bn_relu(p_ref, degp_ref, b_ref, g_ref, be_ref)
    gi = lax.broadcasted_iota(jnp.int32, (_G, 1), 0)
    oh = (batch_ref[...] == gi).astype(jnp.float32)
    pooled_sum = jax.lax.dot_general(
        oh, h, (((1,), (0,)), ((), ())),
        preferred_element_type=jnp.float32)
    counts = jnp.sum(oh, axis=1, keepdims=True)
    pooled = pooled_sum / jnp.maximum(counts, 1.0)
    z = jnp.maximum(
        jnp.dot(pooled, fw1_ref[...], preferred_element_type=jnp.float32)
        + fb1_ref[...], 0.0)
    o_ref[...] = jnp.dot(z, fw2_ref[...],
                         preferred_element_type=jnp.float32) + fb2_ref[...]


_tc_head = pl.pallas_call(
    _tc_head_body,
    out_shape=jax.ShapeDtypeStruct((_G, _C), jnp.float32),
)


def kernel(x, edge_index, batch, W1, b1, g1, be1, W2, b2, g2, be2,
           W3, b3, g3, be3, fW1, fb1, fW2, fb2):
    f32 = jnp.float32
    loop = jnp.arange(_N, dtype=edge_index.dtype)
    pad = _EPAD - _EP
    src = jnp.concatenate([edge_index[0], loop,
                           jnp.full((pad,), _N, edge_index.dtype)])
    dst = jnp.concatenate([edge_index[1], loop,
                           jnp.full((pad,), _N, edge_index.dtype)])
    src_p = src.reshape(_NC, _NS, _NPH, _NITER, _K)
    dst_p = dst.reshape(_NC, _NS, _NPH, _NITER, _K)

    zeros_d = jnp.zeros((_NPAD, _D), f32)
    ones_d = jnp.ones((_K, _D), f32)
    xp = jnp.pad(x, ((0, _NPAD - _N), (0, 0)))
    batch_p = jnp.pad(batch, (0, _NPAD - _N),
                      constant_values=_G).reshape(1, _NPAD)
    row = lambda v: v.reshape(1, -1)

    degp = _deg_kernel(dst_p, zeros_d, ones_d)
    hm1 = _tc_matmul(xp, W1)
    hs1 = _tc_scale(hm1, degp)
    p1 = _msg_kernel(hs1, src_p, dst_p, zeros_d)
    hs2 = _tc_layer(p1, degp, row(b1), row(g1), row(be1), W2)
    p2 = _msg_kernel(hs2, src_p, dst_p, zeros_d)
    hs3 = _tc_layer(p2, degp, row(b2), row(g2), row(be2), W3)
    p3 = _msg_kernel(hs3, src_p, dst_p, zeros_d)
    return _tc_head(p3, degp, row(b3), row(g3), row(be3), batch_p,
                    fW1, row(fb1), fW2, row(fb2))


# speedup vs baseline: 4.4821x; 4.4821x over previous
import functools

import jax
import jax.numpy as jnp
from jax import lax
from jax.experimental import pallas as pl
from jax.experimental.pallas import tpu as pltpu
from jax.experimental.pallas import tpu_sc as plsc

_N = 10000
_E = 320000
_D = 128
_G = 64
_C = 16

_NC = 2
_NS = 16
_K = 128

_NPAD = 10112
_RPS = _NPAD // _NS

_EP = _E + _N
_NPH = 1
_NITER = 84
_EPAD = _NC * _NS * _NPH * _NITER * _K

_mesh = plsc.VectorSubcoreMesh(core_axis_name="c", subcore_axis_name="s")


@functools.partial(
    pl.kernel,
    out_type=jax.ShapeDtypeStruct((_NC, _NPAD, _D), jnp.float32),
    mesh=_mesh,
    scratch_types=[
        pltpu.VMEM((_NITER, _K), jnp.int32),
        pltpu.VMEM((_K, _D), jnp.float32),
        pltpu.VMEM_SHARED((_NPAD, _D), jnp.float32),
    ],
)
def _deg_kernel(dst_hbm, zeros_hbm, ones_hbm, out_hbm, dst_v, ones_v, acc_sh):
    c = lax.axis_index("c")
    s = lax.axis_index("s")
    rows = pl.ds(s * _RPS, _RPS)
    pltpu.sync_copy(zeros_hbm.at[rows], acc_sh.at[rows])
    pltpu.sync_copy(ones_hbm, ones_v)
    for ph in range(_NPH):
        pltpu.sync_copy(dst_hbm.at[c, s, ph], dst_v)
        if ph == 0:
            plsc.subcore_barrier()

        @pl.loop(0, _NITER)
        def _(j):
            pltpu.sync_copy(ones_v, acc_sh.at[dst_v.at[j]], add=True)

    plsc.subcore_barrier()
    pltpu.sync_copy(acc_sh.at[rows], out_hbm.at[c, rows])


@functools.partial(
    pl.kernel,
    out_type=jax.ShapeDtypeStruct((_NC, _NPAD, _D), jnp.float32),
    mesh=_mesh,
    scratch_types=[
        pltpu.VMEM((_NITER, _K), jnp.int32),
        pltpu.VMEM((_NITER, _K), jnp.int32),
        pltpu.VMEM((_K, _D), jnp.float32),
        pltpu.VMEM((_K, _D), jnp.float32),
        pltpu.VMEM_SHARED((_NPAD, _D), jnp.float32),
        pltpu.SemaphoreType.DMA,
        pltpu.SemaphoreType.DMA,
        pltpu.SemaphoreType.DMA,
        pltpu.SemaphoreType.DMA,
    ],
)
def _msg_kernel(hs_hbm, src_hbm, dst_hbm, zeros_hbm, out_hbm,
                src_v, dst_v, b0, b1, acc_sh, sg0, sg1, ss0, ss1):
    c = lax.axis_index("c")
    s = lax.axis_index("s")
    rows = pl.ds(s * _RPS, _RPS)
    pltpu.sync_copy(zeros_hbm.at[rows], acc_sh.at[rows])

    first = [True]

    def _phase(ph):
        pltpu.sync_copy(src_hbm.at[c, s, ph], src_v)
        pltpu.sync_copy(dst_hbm.at[c, s, ph], dst_v)
        if first[0]:
            plsc.subcore_barrier()
            first[0] = False
        @pl.loop(0, _NITER)
        def _(j):
            pltpu.sync_copy(hs_hbm.at[src_v.at[j]], b0)
            pltpu.sync_copy(b0, acc_sh.at[dst_v.at[j]], add=True)

    for ph in range(_NPH):
        _phase(ph)
    plsc.subcore_barrier()
    pltpu.sync_copy(acc_sh.at[rows], out_hbm.at[c, rows])


def _dinv_from_degp(degp):
    deg = degp[0, :, 0:1] + degp[1, :, 0:1]
    return lax.rsqrt(jnp.maximum(deg, 1.0))


def _tc_matmul_body(x_ref, w_ref, o_ref):
    o_ref[...] = jnp.dot(x_ref[...], w_ref[...],
                         preferred_element_type=jnp.float32)


_tc_matmul = pl.pallas_call(
    _tc_matmul_body,
    out_shape=jax.ShapeDtypeStruct((_NPAD, _D), jnp.float32),
)


def _tc_scale_body(hm_ref, degp_ref, o_ref):
    o_ref[...] = hm_ref[...] * _dinv_from_degp(degp_ref[...])


_tc_scale = pl.pallas_call(
    _tc_scale_body,
    out_shape=jax.ShapeDtypeStruct((_NPAD, _D), jnp.float32),
)


def _bn_relu(p_ref, degp_ref, b_ref, g_ref, be_ref):
    dinv = _dinv_from_degp(degp_ref[...])
    y = (p_ref[0] + p_ref[1]) * dinv + b_ref[...]
    mask = lax.broadcasted_iota(jnp.int32, (_NPAD, 1), 0) < _N
    ym = jnp.where(mask, y, 0.0)
    mu = jnp.sum(ym, axis=0, keepdims=True) * (1.0 / _N)
    d2 = jnp.where(mask, y - mu, 0.0)
    var = jnp.sum(d2 * d2, axis=0, keepdims=True) * (1.0 / _N)
    h = (y - mu) * lax.rsqrt(var + 1e-5) * g_ref[...] + be_ref[...]
    h = jnp.maximum(h, 0.0)
    return jnp.where(mask, h, 0.0), dinv


def _tc_layer_body(p_ref, degp_ref, b_ref, g_ref, be_ref, w_ref, o_ref):
    h, dinv = _bn_relu(p_ref, degp_ref, b_ref, g_ref, be_ref)
    o_ref[...] = jnp.dot(h * dinv, w_ref[...],
                         preferred_element_type=jnp.float32)


_tc_layer = pl.pallas_call(
    _tc_layer_body,
    out_shape=jax.ShapeDtypeStruct((_NPAD, _D), jnp.float32),
)


def _tc_head_body(p_ref, degp_ref, b_ref, g_ref, be_ref, batch_ref,
                  fw1_ref, fb1_ref, fw2_ref, fb2_ref, o_ref):
    h, _ = _bn_relu(p_ref, degp_ref, b_ref, g_ref, be_ref)
    gi = lax.broadcasted_iota(jnp.int32, (_G, 1), 0)
    oh = (batch_ref[...] == gi).astype(jnp.float32)
    pooled_sum = jax.lax.dot_general(
        oh, h, (((1,), (0,)), ((), ())),
        preferred_element_type=jnp.float32)
    counts = jnp.sum(oh, axis=1, keepdims=True)
    pooled = pooled_sum / jnp.maximum(counts, 1.0)
    z = jnp.maximum(
        jnp.dot(pooled, fw1_ref[...], preferred_element_type=jnp.float32)
        + fb1_ref[...], 0.0)
    o_ref[...] = jnp.dot(z, fw2_ref[...],
                         preferred_element_type=jnp.float32) + fb2_ref[...]


_tc_head = pl.pallas_call(
    _tc_head_body,
    out_shape=jax.ShapeDtypeStruct((_G, _C), jnp.float32),
)


def kernel(x, edge_index, batch, W1, b1, g1, be1, W2, b2, g2, be2,
           W3, b3, g3, be3, fW1, fb1, fW2, fb2):
    f32 = jnp.float32
    loop = jnp.arange(_N, dtype=edge_index.dtype)
    pad = _EPAD - _EP
    spare = _N + jnp.arange(pad, dtype=edge_index.dtype) % (_NPAD - _N)
    src = jnp.concatenate([edge_index[0], loop, spare])
    dst = jnp.concatenate([edge_index[1], loop, spare])
    src_p = src.reshape(_NC, _NS, _NPH, _NITER, _K)
    dst_p = dst.reshape(_NC, _NS, _NPH, _NITER, _K)

    zeros_d = jnp.zeros((_NPAD, _D), f32)
    ones_d = jnp.ones((_K, _D), f32)
    xp = jnp.pad(x, ((0, _NPAD - _N), (0, 0)))
    batch_p = jnp.pad(batch, (0, _NPAD - _N),
                      constant_values=_G).reshape(1, _NPAD)
    row = lambda v: v.reshape(1, -1)

    degp = _deg_kernel(dst_p, zeros_d, ones_d)
    hm1 = _tc_matmul(xp, W1)
    hs1 = _tc_scale(hm1, degp)
    p1 = _msg_kernel(hs1, src_p, dst_p, zeros_d)
    hs2 = _tc_layer(p1, degp, row(b1), row(g1), row(be1), W2)
    p2 = _msg_kernel(hs2, src_p, dst_p, zeros_d)
    hs3 = _tc_layer(p2, degp, row(b2), row(g2), row(be2), W3)
    p3 = _msg_kernel(hs3, src_p, dst_p, zeros_d)
    return _tc_head(p3, degp, row(b3), row(g3), row(be3), batch_p,
                    fW1, row(fb1), fW2, row(fb2))


# speedup vs baseline: 5.5766x; 1.2442x over previous
import functools

import jax
import jax.numpy as jnp
from jax import lax
from jax.experimental import pallas as pl
from jax.experimental.pallas import tpu as pltpu
from jax.experimental.pallas import tpu_sc as plsc

_N = 10000
_E = 320000
_D = 128
_G = 64
_C = 16

_NC = 2
_NS = 16
_K = 128

_NPAD = 10112
_RPS = _NPAD // _NS

_EP = _E + _N
_NPH = 2
_NITER = 42
_EPAD = _NC * _NS * _NPH * _NITER * _K

_mesh = plsc.VectorSubcoreMesh(core_axis_name="c", subcore_axis_name="s")


@functools.partial(
    pl.kernel,
    out_type=jax.ShapeDtypeStruct((_NC, _NPAD, _D), jnp.float32),
    mesh=_mesh,
    scratch_types=[
        pltpu.VMEM((_NITER, _K), jnp.int32),
        pltpu.VMEM((_K, _D), jnp.float32),
        pltpu.VMEM_SHARED((_NPAD, _D), jnp.float32),
    ],
)
def _deg_kernel(dst_hbm, zeros_hbm, ones_hbm, out_hbm, dst_v, ones_v, acc_sh):
    c = lax.axis_index("c")
    s = lax.axis_index("s")
    rows = pl.ds(s * _RPS, _RPS)
    pltpu.sync_copy(zeros_hbm.at[rows], acc_sh.at[rows])
    pltpu.sync_copy(ones_hbm, ones_v)
    for ph in range(_NPH):
        pltpu.sync_copy(dst_hbm.at[c, s, ph], dst_v)
        if ph == 0:
            plsc.subcore_barrier()

        @pl.loop(0, _NITER)
        def _(j):
            pltpu.sync_copy(ones_v, acc_sh.at[dst_v.at[j]], add=True)

    plsc.subcore_barrier()
    pltpu.sync_copy(acc_sh.at[rows], out_hbm.at[c, rows])


@functools.partial(
    pl.kernel,
    out_type=jax.ShapeDtypeStruct((_NC, _NPAD, _D), jnp.float32),
    mesh=_mesh,
    scratch_types=[
        pltpu.VMEM((_NITER, _K), jnp.int32),
        pltpu.VMEM((_NITER, _K), jnp.int32),
        pltpu.VMEM((_K, _D), jnp.float32),
        pltpu.VMEM((_K, _D), jnp.float32),
        pltpu.VMEM_SHARED((_NPAD, _D), jnp.float32),
        pltpu.SemaphoreType.DMA,
        pltpu.SemaphoreType.DMA,
        pltpu.SemaphoreType.DMA,
        pltpu.SemaphoreType.DMA,
    ],
)
def _msg_kernel(hs_hbm, src_hbm, dst_hbm, zeros_hbm, out_hbm,
                src_v, dst_v, b0, b1, acc_sh, sg0, sg1, ss0, ss1):
    c = lax.axis_index("c")
    s = lax.axis_index("s")
    rows = pl.ds(s * _RPS, _RPS)
    pltpu.sync_copy(zeros_hbm.at[rows], acc_sh.at[rows])

    first = [True]

    def _phase(ph):
        pltpu.sync_copy(src_hbm.at[c, s, ph], src_v)
        pltpu.sync_copy(dst_hbm.at[c, s, ph], dst_v)
        if first[0]:
            plsc.subcore_barrier()
            first[0] = False
        @pl.loop(0, _NITER, step=2)
        def _(j):
            @pl.when(j > 0)
            def _():
                pltpu.make_async_copy(b0, acc_sh.at[dst_v.at[j]], ss0).wait()
            pltpu.sync_copy(hs_hbm.at[src_v.at[j]], b0)
            pltpu.async_copy(b0, acc_sh.at[dst_v.at[j]], ss0, add=True)

            @pl.when(j > 0)
            def _():
                pltpu.make_async_copy(b1, acc_sh.at[dst_v.at[j]], ss1).wait()
            pltpu.sync_copy(hs_hbm.at[src_v.at[j + 1]], b1)
            pltpu.async_copy(b1, acc_sh.at[dst_v.at[j + 1]], ss1, add=True)

        pltpu.make_async_copy(b0, acc_sh.at[dst_v.at[0]], ss0).wait()
        pltpu.make_async_copy(b1, acc_sh.at[dst_v.at[1]], ss1).wait()

    for ph in range(_NPH):
        _phase(ph)
    plsc.subcore_barrier()
    pltpu.sync_copy(acc_sh.at[rows], out_hbm.at[c, rows])


def _dinv_from_degp(degp):
    deg = degp[0, :, 0:1] + degp[1, :, 0:1]
    return lax.rsqrt(jnp.maximum(deg, 1.0))


def _tc_matmul_body(x_ref, w_ref, o_ref):
    o_ref[...] = jnp.dot(x_ref[...], w_ref[...],
                         preferred_element_type=jnp.float32)


_tc_matmul = pl.pallas_call(
    _tc_matmul_body,
    out_shape=jax.ShapeDtypeStruct((_NPAD, _D), jnp.float32),
)


def _tc_scale_body(hm_ref, degp_ref, o_ref):
    o_ref[...] = hm_ref[...] * _dinv_from_degp(degp_ref[...])


_tc_scale = pl.pallas_call(
    _tc_scale_body,
    out_shape=jax.ShapeDtypeStruct((_NPAD, _D), jnp.float32),
)


def _bn_relu(p_ref, degp_ref, b_ref, g_ref, be_ref):
    dinv = _dinv_from_degp(degp_ref[...])
    y = (p_ref[0] + p_ref[1]) * dinv + b_ref[...]
    mask = lax.broadcasted_iota(jnp.int32, (_NPAD, 1), 0) < _N
    ym = jnp.where(mask, y, 0.0)
    mu = jnp.sum(ym, axis=0, keepdims=True) * (1.0 / _N)
    d2 = jnp.where(mask, y - mu, 0.0)
    var = jnp.sum(d2 * d2, axis=0, keepdims=True) * (1.0 / _N)
    h = (y - mu) * lax.rsqrt(var + 1e-5) * g_ref[...] + be_ref[...]
    h = jnp.maximum(h, 0.0)
    return jnp.where(mask, h, 0.0), dinv


def _tc_layer_body(p_ref, degp_ref, b_ref, g_ref, be_ref, w_ref, o_ref):
    h, dinv = _bn_relu(p_ref, degp_ref, b_ref, g_ref, be_ref)
    o_ref[...] = jnp.dot(h * dinv, w_ref[...],
                         preferred_element_type=jnp.float32)


_tc_layer = pl.pallas_call(
    _tc_layer_body,
    out_shape=jax.ShapeDtypeStruct((_NPAD, _D), jnp.float32),
)


def _tc_head_body(p_ref, degp_ref, b_ref, g_ref, be_ref, batch_ref,
                  fw1_ref, fb1_ref, fw2_ref, fb2_ref, o_ref):
    h, _ = _bn_relu(p_ref, degp_ref, b_ref, g_ref, be_ref)
    gi = lax.broadcasted_iota(jnp.int32, (_G, 1), 0)
    oh = (batch_ref[...] == gi).astype(jnp.float32)
    pooled_sum = jax.lax.dot_general(
        oh, h, (((1,), (0,)), ((), ())),
        preferred_element_type=jnp.float32)
    counts = jnp.sum(oh, axis=1, keepdims=True)
    pooled = pooled_sum / jnp.maximum(counts, 1.0)
    z = jnp.maximum(
        jnp.dot(pooled, fw1_ref[...], preferred_element_type=jnp.float32)
        + fb1_ref[...], 0.0)
    o_ref[...] = jnp.dot(z, fw2_ref[...],
                         preferred_element_type=jnp.float32) + fb2_ref[...]


_tc_head = pl.pallas_call(
    _tc_head_body,
    out_shape=jax.ShapeDtypeStruct((_G, _C), jnp.float32),
)


def kernel(x, edge_index, batch, W1, b1, g1, be1, W2, b2, g2, be2,
           W3, b3, g3, be3, fW1, fb1, fW2, fb2):
    f32 = jnp.float32
    loop = jnp.arange(_N, dtype=edge_index.dtype)
    pad = _EPAD - _EP
    spare = _N + jnp.arange(pad, dtype=edge_index.dtype) % (_NPAD - _N)
    src = jnp.concatenate([edge_index[0], loop, spare])
    dst = jnp.concatenate([edge_index[1], loop, spare])
    src_p = src.reshape(_NC, _NS, _NPH, _NITER, _K)
    dst_p = dst.reshape(_NC, _NS, _NPH, _NITER, _K)

    zeros_d = jnp.zeros((_NPAD, _D), f32)
    ones_d = jnp.ones((_K, _D), f32)
    xp = jnp.pad(x, ((0, _NPAD - _N), (0, 0)))
    batch_p = jnp.pad(batch, (0, _NPAD - _N),
                      constant_values=_G).reshape(1, _NPAD)
    row = lambda v: v.reshape(1, -1)

    degp = _deg_kernel(dst_p, zeros_d, ones_d)
    hm1 = _tc_matmul(xp, W1)
    hs1 = _tc_scale(hm1, degp)
    p1 = _msg_kernel(hs1, src_p, dst_p, zeros_d)
    hs2 = _tc_layer(p1, degp, row(b1), row(g1), row(be1), W2)
    p2 = _msg_kernel(hs2, src_p, dst_p, zeros_d)
    hs3 = _tc_layer(p2, degp, row(b2), row(g2), row(be2), W3)
    p3 = _msg_kernel(hs3, src_p, dst_p, zeros_d)
    return _tc_head(p3, degp, row(b3), row(g3), row(be3), batch_p,
                    fW1, row(fb1), fW2, row(fb2))
